# baseline (speedup 1.0000x reference)
import functools

import jax
import jax.numpy as jnp
from jax.experimental import pallas as pl
from jax.experimental.pallas import tpu as pltpu

N = 50000
E = 800000
G = 1024
D_IN = 5
H = 64


def _mlp_body(p_ref, Wf1_ref, bf1_ref, Wf2_ref, bf2_ref, Wf3_ref, bf3_ref, o_ref):
    p = p_ref[...]
    h = jnp.maximum(jnp.dot(p, Wf1_ref[...], preferred_element_type=jnp.float32) + bf1_ref[...], 0.0)
    h = jnp.maximum(jnp.dot(h, Wf2_ref[...], preferred_element_type=jnp.float32) + bf2_ref[...], 0.0)
    o_ref[...] = jnp.dot(h, Wf3_ref[...], preferred_element_type=jnp.float32) + bf3_ref[...]


def _mlp_head(p, Wf1, bf1, Wf2, bf2, Wf3, bf3):
    return pl.pallas_call(
        _mlp_body,
        out_shape=jax.ShapeDtypeStruct((G, 1), jnp.float32),
    )(p, Wf1, bf1[None, :], Wf2, bf2[None, :], Wf3, bf3[None, :])


def _gat_jnp(h_in, src, dst, W, a_src, a_dst, b):
    h = h_in @ W
    alpha_s = jnp.sum(h * a_src, axis=-1)
    alpha_d = jnp.sum(h * a_dst, axis=-1)
    e = jax.nn.leaky_relu(jnp.take(alpha_s, src) + jnp.take(alpha_d, dst), negative_slope=0.2)
    m = jax.ops.segment_max(e, dst, num_segments=N)
    m = jnp.where(jnp.isfinite(m), m, 0.0)
    ex = jnp.exp(e - jnp.take(m, dst))
    den = jax.ops.segment_sum(ex, dst, num_segments=N)
    alpha = ex / (jnp.take(den, dst) + 1e-16)
    out = jax.ops.segment_sum(jnp.take(h, src, axis=0) * alpha[:, None], dst, num_segments=N)
    return out + b


def kernel(x, edge_index, batch, W_embed, b_embed, W1, a_src1, a_dst1, b1,
           W2, a_src2, a_dst2, b2, Wf1, bf1, Wf2, bf2, Wf3, bf3):
    loops = jnp.arange(N, dtype=edge_index.dtype)
    src = jnp.concatenate([edge_index[0], loops])
    dst = jnp.concatenate([edge_index[1], loops])
    h = x @ W_embed + b_embed
    h = _gat_jnp(h, src, dst, W1, a_src1, a_dst1, b1)
    h = jax.nn.relu(h)
    h = _gat_jnp(h, src, dst, W2, a_src2, a_dst2, b2)
    s = jax.ops.segment_sum(h, batch, num_segments=G)
    cnt = jax.ops.segment_sum(jnp.ones((N,), jnp.float32), batch, num_segments=G)
    mean = s / jnp.maximum(cnt, 1.0)[:, None]
    mx = jax.ops.segment_max(h, batch, num_segments=G)
    mx = jnp.where(jnp.isfinite(mx), mx, 0.0)
    p = jnp.concatenate([mean, mx], axis=1)
    return _mlp_head(p, Wf1, bf1, Wf2, bf2, Wf3, bf3)


# speedup vs baseline: 7.8228x; 7.8228x over previous
import functools

import jax
import jax.numpy as jnp
from jax import lax
from jax.experimental import pallas as pl
from jax.experimental.pallas import tpu as pltpu
from jax.experimental.pallas import tpu_sc as plsc

N = 50000
E = 800000
G = 1024
H = 64
HW = 32

NHALF = 25000
NHP = NHALF + 88
W48 = 48
CHUNK = 128
NTILES = 16
ROWS_PER_TILE = NHP // NTILES
NEG_BIG = -3.0e38

EPAD = 416 * CHUNK * NTILES
EPAD_P = 25 * CHUNK * NTILES

BR = 400
NB = N // BR

_SC_PARAMS = pltpu.CompilerParams(
    needs_layout_passes=False, use_tc_tiling_on_sc=False)


def _sc_mesh():
    return plsc.VectorSubcoreMesh(core_axis_name="c", subcore_axis_name="s")


def _make_ex_kernel(e_tot, nch):
    @functools.partial(
        pl.kernel,
        mesh=_sc_mesh(),
        compiler_params=_SC_PARAMS,
        out_type=jax.ShapeDtypeStruct((nch * CHUNK,), jnp.float32),
        scratch_types=[
            pltpu.VMEM((N,), jnp.float32),
            pltpu.VMEM((N,), jnp.float32),
            pltpu.VMEM((16,), jnp.float32),
            pltpu.VMEM((CHUNK,), jnp.int32),
            pltpu.VMEM((CHUNK,), jnp.int32),
            pltpu.VMEM((CHUNK,), jnp.float32),
        ],
    )
    def k(as_hbm, ad_hbm, src_hbm, dst_hbm, mb_hbm, ex_hbm,
          as_v, ad_v, mb_v, src_v, dst_v, exo_v):
        c = lax.axis_index("c")
        s = lax.axis_index("s")
        wid = s * 2 + c
        lane = lax.iota(jnp.int32, 16)

        pltpu.sync_copy(as_hbm, as_v)
        pltpu.sync_copy(ad_hbm, ad_v)
        pltpu.sync_copy(mb_hbm.at[0, pl.ds(0, 16)], mb_v)
        mbv = mb_v[...]

        def chunk_body(jj, _):
            eb = (jj * 32 + wid) * CHUNK
            pltpu.sync_copy(src_hbm.at[pl.ds(eb, CHUNK)], src_v)
            pltpu.sync_copy(dst_hbm.at[pl.ds(eb, CHUNK)], dst_v)
            for v in range(CHUNK // 16):
                sl = pl.ds(v * 16, 16)
                sv = plsc.load_gather(as_v, [src_v[sl]])
                dv = plsc.load_gather(ad_v, [dst_v[sl]])
                e = sv + dv
                e = jnp.where(e >= 0.0, e, e * 0.2)
                ex = jnp.exp(e - mbv)
                eid = eb + v * 16 + lane
                exo_v[sl] = jnp.where(eid < e_tot, ex, 0.0)
            pltpu.sync_copy(exo_v, ex_hbm.at[pl.ds(eb, CHUNK)])
            return 0
        lax.fori_loop(0, nch // 32, chunk_body, 0)

    return k


_ex_gat = _make_ex_kernel(E + N, EPAD // CHUNK)


def _make_edge_kernel(cpt):
    @functools.partial(
        pl.kernel,
        mesh=_sc_mesh(),
        compiler_params=_SC_PARAMS,
        out_type=jax.ShapeDtypeStruct((2, NHP, W48), jnp.float32),
        scratch_types=[
            pltpu.VMEM((CHUNK,), jnp.int32),
            pltpu.VMEM((CHUNK,), jnp.int32),
            pltpu.VMEM((CHUNK,), jnp.int32),
            pltpu.VMEM((CHUNK,), jnp.float32),
            pltpu.VMEM((CHUNK, HW), jnp.float32),
            pltpu.VMEM((CHUNK, W48), jnp.float32),
            pltpu.VMEM_SHARED((NHP, W48), jnp.float32),
            pltpu.SemaphoreType.DMA,
        ],
    )
    def k(h_hbm, src_hbm, dst_hbm, ex_hbm, num_hbm,
          src_v, dst_v, loc_v, ex_v, rows_v, out_v, acc, sem):
        c = lax.axis_index("c")
        s = lax.axis_index("s")
        lane = lax.iota(jnp.int32, 16)
        zeros16 = jnp.zeros((16,), jnp.float32)

        def zrow(r, _):
            for j in range(W48 // 16):
                out_v[r, pl.ds(16 * j, 16)] = zeros16
            return 0
        lax.fori_loop(0, CHUNK, zrow, 0)

        r0 = s * ROWS_PER_TILE
        nfull = ROWS_PER_TILE // CHUNK
        rem = ROWS_PER_TILE - nfull * CHUNK

        def zacc(i, _):
            pltpu.sync_copy(out_v, acc.at[pl.ds(r0 + i * CHUNK, CHUNK)])
            return 0
        lax.fori_loop(0, nfull, zacc, 0)
        pltpu.sync_copy(out_v.at[pl.ds(0, rem)],
                        acc.at[pl.ds(r0 + nfull * CHUNK, rem)])
        plsc.subcore_barrier()

        lo = c * NHALF

        def chunk_body(kk, _):
            eb = (s * cpt + kk) * CHUNK
            pltpu.sync_copy(src_hbm.at[pl.ds(eb, CHUNK)], src_v)
            pltpu.sync_copy(dst_hbm.at[pl.ds(eb, CHUNK)], dst_v)
            pltpu.sync_copy(ex_hbm.at[pl.ds(eb, CHUNK)], ex_v)
            pltpu.async_copy(h_hbm.at[src_v], rows_v, sem).wait()
            for v in range(CHUNK // 16):
                sl = pl.ds(v * 16, 16)
                di = dst_v[sl]
                ok = (di >= lo) & (di < lo + NHALF)
                loc_v[sl] = jnp.where(ok, di - lo, NHALF)

            def scale(r, _):
                exv = plsc.load_gather(ex_v, [jnp.full((16,), r, jnp.int32)])
                for j in range(HW // 16):
                    out_v[r, pl.ds(16 * j, 16)] = (
                        rows_v[r, pl.ds(16 * j, 16)] * exv)
                out_v[r, pl.ds(HW, 16)] = jnp.where(lane == 0, exv, 0.0)
                return 0
            lax.fori_loop(0, CHUNK, scale, 0)
            pltpu.sync_copy(out_v, acc.at[loc_v], add=True)
            return 0
        lax.fori_loop(0, cpt, chunk_body, 0)
        plsc.subcore_barrier()

        def wb(i, _):
            pltpu.sync_copy(acc.at[pl.ds(r0 + i * CHUNK, CHUNK)],
                            num_hbm.at[c, pl.ds(r0 + i * CHUNK, CHUNK)])
            return 0
        lax.fori_loop(0, nfull, wb, 0)
        pltpu.sync_copy(acc.at[pl.ds(r0 + nfull * CHUNK, rem)],
                        num_hbm.at[c, pl.ds(r0 + nfull * CHUNK, rem)])

    return k


_edge_gat = _make_edge_kernel(EPAD // CHUNK // NTILES)
_edge_pool = _make_edge_kernel(EPAD_P // CHUNK // NTILES)

NP_PAD = 48
NCHUNKS_P = (N + NP_PAD) // CHUNK


def _make_pool_max_parts():
    @functools.partial(
        pl.kernel,
        mesh=_sc_mesh(),
        compiler_params=_SC_PARAMS,
        out_type=jax.ShapeDtypeStruct((32, G, H), jnp.float32),
        scratch_types=[
            pltpu.VMEM((CHUNK, HW), jnp.float32),
            pltpu.VMEM((CHUNK, HW), jnp.float32),
            pltpu.VMEM((CHUNK,), jnp.int32),
            pltpu.VMEM((G, H), jnp.float32),
        ],
    )
    def k(ha_hbm, hb_hbm, batch_hbm, parts_hbm, ha_v, hb_v, b_v, acc_v):
        c = lax.axis_index("c")
        s = lax.axis_index("s")
        wid = s * 2 + c
        neg = jnp.full((16,), NEG_BIG, jnp.float32)

        def irow(g, _):
            for j in range(4):
                acc_v[g, pl.ds(16 * j, 16)] = neg
            return 0
        lax.fori_loop(0, G, irow, 0)

        def chunk_body(jj, _):
            nb = (jj * 32 + wid) * CHUNK
            nrows = jnp.minimum(CHUNK, N - nb)
            pltpu.sync_copy(ha_hbm.at[pl.ds(nb, CHUNK)], ha_v)
            pltpu.sync_copy(hb_hbm.at[pl.ds(nb, CHUNK)], hb_v)
            pltpu.sync_copy(batch_hbm.at[pl.ds(nb, CHUNK)], b_v)

            def pgrp(v, _):
                bv16 = b_v[pl.ds(v * 16, 16)]
                for t in range(16):
                    g = bv16[t]
                    r = v * 16 + t
                    for j2 in range(2):
                        sl = pl.ds(16 * j2, 16)
                        acc_v[g, sl] = jnp.maximum(acc_v[g, sl], ha_v[r, sl])
                        sl2 = pl.ds(HW + 16 * j2, 16)
                        acc_v[g, sl2] = jnp.maximum(acc_v[g, sl2],
                                                    hb_v[r, sl])
                return 0
            lax.fori_loop(0, nrows // 16, pgrp, 0)
            return 0
        nch = (NCHUNKS_P - 1 - wid) // 32 + 1
        lax.fori_loop(0, nch, chunk_body, 0)

        pltpu.sync_copy(acc_v, parts_hbm.at[wid])

    return k


def _make_pool_max_reduce():
    @functools.partial(
        pl.kernel,
        mesh=_sc_mesh(),
        compiler_params=_SC_PARAMS,
        out_type=jax.ShapeDtypeStruct((G, H), jnp.float32),
        scratch_types=[
            pltpu.VMEM((32, H), jnp.float32),
            pltpu.VMEM((32, H), jnp.float32),
        ],
    )
    def k(parts_hbm, out_hbm, tmp_v, res_v):
        c = lax.axis_index("c")
        s = lax.axis_index("s")
        wid = s * 2 + c
        g0 = wid * (G // 32)
        neg = jnp.full((16,), NEG_BIG, jnp.float32)

        def irow(r, _):
            for j in range(4):
                res_v[r, pl.ds(16 * j, 16)] = neg
            return 0
        lax.fori_loop(0, 32, irow, 0)

        def comb(t, _):
            pltpu.sync_copy(parts_hbm.at[t, pl.ds(g0, 32)], tmp_v)

            def mrow(r, _):
                for j in range(4):
                    sl = pl.ds(16 * j, 16)
                    res_v[r, sl] = jnp.maximum(res_v[r, sl], tmp_v[r, sl])
                return 0
            lax.fori_loop(0, 32, mrow, 0)
            return 0
        lax.fori_loop(0, 32, comb, 0)
        pltpu.sync_copy(res_v, out_hbm.at[pl.ds(g0, 32)])

    return k


_pool_max_parts = _make_pool_max_parts()
_pool_max_reduce = _make_pool_max_reduce()


def _tc_a_body(x_ref, wep_ref, be_ref, w1_ref, as1_ref, ad1_ref,
               ha_ref, hb_ref, aso_ref, ado_ref, mb_ref, ms_s, md_s):
    i = pl.program_id(0)

    @pl.when(i == 0)
    def _():
        ms_s[0] = NEG_BIG
        md_s[0] = NEG_BIG

    h0 = jnp.dot(x_ref[...], wep_ref[...],
                 preferred_element_type=jnp.float32) + be_ref[...]
    h1 = jnp.dot(h0, w1_ref[...], preferred_element_type=jnp.float32)
    ha_ref[...] = h1[:, :HW]
    hb_ref[...] = h1[:, HW:]
    a_s = jnp.sum(h1 * as1_ref[...], axis=1)
    a_d = jnp.sum(h1 * ad1_ref[...], axis=1)
    aso_ref[...] = a_s[None, None, :]
    ado_ref[...] = a_d[None, None, :]
    ms_s[0] = jnp.maximum(ms_s[0], jnp.max(a_s))
    md_s[0] = jnp.maximum(md_s[0], jnp.max(a_d))

    @pl.when(i == NB - 1)
    def _():
        mb_ref[...] = jnp.zeros((8, 128), jnp.float32) + jnp.maximum(
            ms_s[0] + md_s[0], 0.0)


def _h_outs():
    return (
        [
            pl.BlockSpec((BR, HW), lambda i: (i, 0)),
            pl.BlockSpec((BR, HW), lambda i: (i, 0)),
            pl.BlockSpec((1, 1, BR), lambda i: (i, 0, 0)),
            pl.BlockSpec((1, 1, BR), lambda i: (i, 0, 0)),
            pl.BlockSpec((8, 128), lambda i: (0, 0)),
        ],
        [
            jax.ShapeDtypeStruct((N, HW), jnp.float32),
            jax.ShapeDtypeStruct((N, HW), jnp.float32),
            jax.ShapeDtypeStruct((NB, 1, BR), jnp.float32),
            jax.ShapeDtypeStruct((NB, 1, BR), jnp.float32),
            jax.ShapeDtypeStruct((8, 128), jnp.float32),
        ],
    )


def _tc_a(xp, wep, be, w1, as1, ad1):
    out_specs, out_shape = _h_outs()
    return pl.pallas_call(
        _tc_a_body,
        grid=(NB,),
        in_specs=[
            pl.BlockSpec((BR, 8), lambda i: (i, 0)),
            pl.BlockSpec((8, H), lambda i: (0, 0)),
            pl.BlockSpec((1, H), lambda i: (0, 0)),
            pl.BlockSpec((H, H), lambda i: (0, 0)),
            pl.BlockSpec((1, H), lambda i: (0, 0)),
            pl.BlockSpec((1, H), lambda i: (0, 0)),
        ],
        out_specs=out_specs,
        out_shape=out_shape,
        scratch_shapes=[
            pltpu.SMEM((1,), jnp.float32),
            pltpu.SMEM((1,), jnp.float32),
        ],
    )(xp, wep, be, w1, as1, ad1)


def _tc_b_body(na_ref, nb_ref, b1_ref, w2_ref, as2_ref, ad2_ref,
               ha_ref, hb_ref, aso_ref, ado_ref, mb_ref, ms_s, md_s):
    i = pl.program_id(0)

    @pl.when(i == 0)
    def _():
        ms_s[0] = NEG_BIG
        md_s[0] = NEG_BIG

    na = na_ref[...]
    nb = nb_ref[...]
    den = na[:, HW:HW + 1] + 1e-16
    out1 = jnp.concatenate([na[:, :HW] / den, nb[:, :HW] / den], axis=1)
    hr = jnp.maximum(out1 + b1_ref[...], 0.0)
    h2 = jnp.dot(hr, w2_ref[...], preferred_element_type=jnp.float32)
    ha_ref[...] = h2[:, :HW]
    hb_ref[...] = h2[:, HW:]
    a_s = jnp.sum(h2 * as2_ref[...], axis=1)
    a_d = jnp.sum(h2 * ad2_ref[...], axis=1)
    aso_ref[...] = a_s[None, None, :]
    ado_ref[...] = a_d[None, None, :]
    ms_s[0] = jnp.maximum(ms_s[0], jnp.max(a_s))
    md_s[0] = jnp.maximum(md_s[0], jnp.max(a_d))

    @pl.when(i == NB - 1)
    def _():
        mb_ref[...] = jnp.zeros((8, 128), jnp.float32) + jnp.maximum(
            ms_s[0] + md_s[0], 0.0)


def _tc_b(naf, nbf, b1, w2, as2, ad2):
    out_specs, out_shape = _h_outs()
    return pl.pallas_call(
        _tc_b_body,
        grid=(NB,),
        in_specs=[
            pl.BlockSpec((BR, W48), lambda i: (i, 0)),
            pl.BlockSpec((BR, W48), lambda i: (i, 0)),
            pl.BlockSpec((1, H), lambda i: (0, 0)),
            pl.BlockSpec((H, H), lambda i: (0, 0)),
            pl.BlockSpec((1, H), lambda i: (0, 0)),
            pl.BlockSpec((1, H), lambda i: (0, 0)),
        ],
        out_specs=out_specs,
        out_shape=out_shape,
        scratch_shapes=[
            pltpu.SMEM((1,), jnp.float32),
            pltpu.SMEM((1,), jnp.float32),
        ],
    )(naf, nbf, b1, w2, as2, ad2)


def _tc_c_body(na_ref, nb_ref, b2_ref, ha_ref, hb_ref):
    na = na_ref[...]
    nb = nb_ref[...]
    den = na[:, HW:HW + 1] + 1e-16
    b2 = b2_ref[...]
    ha_ref[...] = na[:, :HW] / den + b2[:, :HW]
    hb_ref[...] = nb[:, :HW] / den + b2[:, HW:]


def _tc_c(naf, nbf, b2):
    return pl.pallas_call(
        _tc_c_body,
        grid=(NB,),
        in_specs=[
            pl.BlockSpec((BR, W48), lambda i: (i, 0)),
            pl.BlockSpec((BR, W48), lambda i: (i, 0)),
            pl.BlockSpec((1, H), lambda i: (0, 0)),
        ],
        out_specs=[
            pl.BlockSpec((BR, HW), lambda i: (i, 0)),
            pl.BlockSpec((BR, HW), lambda i: (i, 0)),
        ],
        out_shape=[
            jax.ShapeDtypeStruct((N, HW), jnp.float32),
            jax.ShapeDtypeStruct((N, HW), jnp.float32),
        ],
    )(naf, nbf, b2)


def _tc_d_body(sa_ref, sb_ref, mx_ref, wf1_ref, bf1_ref, wf2_ref, bf2_ref,
               wf3_ref, bf3_ref, o_ref):
    sa = sa_ref[...]
    sb = sb_ref[...]
    cnt = sa[:, HW:HW + 1]
    den = jnp.maximum(cnt, 1.0)
    mean = jnp.concatenate([sa[:, :HW] / den, sb[:, :HW] / den], axis=1)
    mx = jnp.where(cnt > 0.0, mx_ref[...], 0.0)
    p = jnp.concatenate([mean, mx], axis=1)
    h = jnp.maximum(jnp.dot(p, wf1_ref[...],
                            preferred_element_type=jnp.float32)
                    + bf1_ref[...], 0.0)
    h = jnp.maximum(jnp.dot(h, wf2_ref[...],
                            preferred_element_type=jnp.float32)
                    + bf2_ref[...], 0.0)
    o_ref[...] = jnp.dot(h, wf3_ref[...],
                         preferred_element_type=jnp.float32) + bf3_ref[...]


def _tc_d(sa, sb, mx, wf1, bf1, wf2, bf2, wf3, bf3):
    return pl.pallas_call(
        _tc_d_body,
        out_shape=jax.ShapeDtypeStruct((G, 1), jnp.float32),
    )(sa, sb, mx, wf1, bf1, wf2, bf2, wf3, bf3)


def _halves(num):
    return jnp.concatenate([num[0, :NHALF], num[1, :NHALF]], axis=0)


def kernel(x, edge_index, batch, W_embed, b_embed, W1, a_src1, a_dst1, b1,
           W2, a_src2, a_dst2, b2, Wf1, bf1, Wf2, bf2, Wf3, bf3):
    loops = jnp.arange(N, dtype=jnp.int32)
    e_tot = E + N
    src_e = jnp.concatenate([edge_index[0].astype(jnp.int32), loops])
    dst_e = jnp.concatenate([edge_index[1].astype(jnp.int32), loops])
    src_e = jnp.pad(src_e, (0, EPAD - e_tot))
    dst_e = jnp.pad(dst_e, (0, EPAD - e_tot))
    xp = jnp.pad(x, ((0, 0), (0, 3)))
    wep = jnp.pad(W_embed, ((0, 3), (0, 0)))

    h1a, h1b, aso1, ado1, mb1 = _tc_a(xp, wep, b_embed[None], W1,
                                      a_src1[None], a_dst1[None])
    ex1 = _ex_gat(aso1.reshape(N), ado1.reshape(N), src_e, dst_e, mb1)
    n1a = _edge_gat(h1a, src_e, dst_e, ex1)
    n1b = _edge_gat(h1b, src_e, dst_e, ex1)

    h2a, h2b, aso2, ado2, mb2 = _tc_b(_halves(n1a), _halves(n1b), b1[None],
                                      W2, a_src2[None], a_dst2[None])
    ex2 = _ex_gat(aso2.reshape(N), ado2.reshape(N), src_e, dst_e, mb2)
    n2a = _edge_gat(h2a, src_e, dst_e, ex2)
    n2b = _edge_gat(h2b, src_e, dst_e, ex2)

    hfa, hfb = _tc_c(_halves(n2a), _halves(n2b), b2[None])

    srcp = jnp.pad(loops, (0, EPAD_P - N))
    dstp = jnp.pad(batch.astype(jnp.int32), (0, EPAD_P - N))
    exp_ones = jnp.pad(jnp.ones((N,), jnp.float32), (0, EPAD_P - N))
    sa = _edge_pool(hfa, srcp, dstp, exp_ones)
    sb = _edge_pool(hfb, srcp, dstp, exp_ones)

    bp = jnp.pad(batch.astype(jnp.int32), (0, NP_PAD))
    hpa = jnp.pad(hfa, ((0, NP_PAD), (0, 0)))
    hpb = jnp.pad(hfb, ((0, NP_PAD), (0, 0)))
    parts = _pool_max_parts(hpa, hpb, bp)
    mx = _pool_max_reduce(parts)

    return _tc_d(sa[0, :G], sb[0, :G], mx, Wf1, bf1[None], Wf2, bf2[None],
                 Wf3, bf3[None])
